# baseline (speedup 1.0000x reference)
import functools

import jax
import jax.numpy as jnp
from jax import lax
from jax.experimental import pallas as pl
from jax.experimental.pallas import tpu as pltpu
from jax.experimental.pallas import tpu_sc as plsc

NC = 2
NS = 16
LN = 16
CH = 80
AUGW = 272

_f32 = jnp.float32
_bf16 = jnp.bfloat16
_i32 = jnp.int32


HWW = 288


def _aug(cat, d):
    r = cat.shape[0]
    ones = jnp.ones((r, 1), _f32)
    pad = jnp.zeros((r, HWW - d - 1), _f32)
    return jnp.concatenate([cat[:, d:], ones, pad], axis=1).astype(_bf16)


def _agg_mean(agg_ref, d):
    a = agg_ref[0].astype(_f32) + agg_ref[1].astype(_f32)
    rdeg = 1.0 / jnp.maximum(a[:, d:d + 1], 1.0)
    return a[:, :d] * rdeg


def _mm1_body(x_ref, w_ref, hs_ref, hw_ref):
    cat = jnp.dot(x_ref[...], w_ref[...], preferred_element_type=_f32)
    d = x_ref.shape[1]
    hs_ref[...] = cat[:, :d]
    hw_ref[...] = _aug(cat, d)


def _mm2_body(hs_ref, agg_ref, b_ref, w_ref, hs2_ref, hw2_ref):
    d = hs_ref.shape[1]
    h1 = jnp.maximum(hs_ref[...] + _agg_mean(agg_ref, d) + b_ref[...], 0.0)
    cat = jnp.dot(h1, w_ref[...], preferred_element_type=_f32)
    hs2_ref[...] = cat[:, :d]
    hw2_ref[...] = _aug(cat, d)


def _comb_body(hs_ref, agg_ref, b_ref, out_ref):
    d = hs_ref.shape[1]
    h2 = hs_ref[...] + _agg_mean(agg_ref, d) + b_ref[...]
    nsq = jnp.sum(h2 * h2, axis=1, keepdims=True)
    pad = jnp.zeros((h2.shape[0], AUGW - h2.shape[1] - 1), _f32)
    out_ref[...] = jnp.concatenate([h2, nsq, pad], axis=1)


def _tc_specs(d, r):
    row = pl.BlockSpec((r, d), lambda i: (i, 0))
    hww = pl.BlockSpec((r, HWW), lambda i: (i, 0))
    aggs = pl.BlockSpec((2, r, HWW), lambda i: (0, i, 0))
    bias = pl.BlockSpec((1, d), lambda i: (0, 0))
    wmat = pl.BlockSpec((d, 2 * d), lambda i: (0, 0))
    return row, hww, aggs, bias, wmat


def _mm1(x, wcat, r):
    n, d = x.shape
    row, hww, _, _, wmat = _tc_specs(d, r)
    return pl.pallas_call(
        _mm1_body,
        grid=(n // r,),
        in_specs=[row, wmat],
        out_specs=[row, hww],
        out_shape=[jax.ShapeDtypeStruct((n, d), _f32),
                   jax.ShapeDtypeStruct((n, HWW), _bf16)],
    )(x, wcat)


def _mm2(hs, agg, b2, wcat, r):
    n, d = hs.shape
    row, hww, aggs, bias, wmat = _tc_specs(d, r)
    return pl.pallas_call(
        _mm2_body,
        grid=(n // r,),
        in_specs=[row, aggs, bias, wmat],
        out_specs=[row, hww],
        out_shape=[jax.ShapeDtypeStruct((n, d), _f32),
                   jax.ShapeDtypeStruct((n, HWW), _bf16)],
    )(hs, agg, b2, wcat)


def _comb(hs, agg, b2, r):
    n, d = hs.shape
    row, hww, aggs, bias, _ = _tc_specs(d, r)
    return pl.pallas_call(
        _comb_body,
        grid=(n // r,),
        in_specs=[row, aggs, bias],
        out_specs=pl.BlockSpec((r, AUGW), lambda i: (i, 0)),
        out_shape=jax.ShapeDtypeStruct((n, AUGW), _f32),
    )(hs, agg, b2)


def _make_agg(n, e):
    w = HWW
    CHA = 40
    npad = ((n + 127) // 128) * 128
    rows_pt = npad // NS
    e_pt = e // NC // NS
    nchunks = e_pt // CHA
    assert e_pt % CHA == 0

    mesh = plsc.VectorSubcoreMesh(core_axis_name="c", subcore_axis_name="s")

    NB = 3
    assert (nchunks - NB - 2) % NB == 0 and nchunks >= 2 * NB

    scratch = [
        *[pltpu.VMEM((CHA,), _i32) for _ in range(NB)],
        *[pltpu.VMEM((CHA,), _i32) for _ in range(NB)],
        *[pltpu.VMEM((CHA, w), _bf16) for _ in range(NB)],
        *[pltpu.SemaphoreType.DMA for _ in range(4 * NB)],
        pltpu.VMEM_SHARED((npad, w), _bf16),
    ]

    def body(hw_ref, src_ref, dst_ref, z_ref, agg_out, *rest):
        idxs = rest[0:NB]
        idxd = rest[NB:2 * NB]
        rows = rest[2 * NB:3 * NB]
        gsem = rest[3 * NB:4 * NB]
        isem = rest[4 * NB:5 * NB]
        dsem = rest[5 * NB:6 * NB]
        ssem = rest[6 * NB:7 * NB]
        agg_sh = rest[7 * NB]

        c = lax.axis_index("c")
        s = lax.axis_index("s")
        r0 = s * rows_pt
        ebase = c * (e // NC) + s * e_pt

        pltpu.sync_copy(z_ref, agg_sh.at[pl.ds(r0, rows_pt)])
        plsc.subcore_barrier()

        def idxload(k, p):
            pltpu.async_copy(src_ref.at[pl.ds(ebase + k * CHA, CHA)],
                             idxs[p], isem[p])
            pltpu.async_copy(dst_ref.at[pl.ds(ebase + k * CHA, CHA)],
                             idxd[p], dsem[p])

        def slot(k, p, load_next=True, scat=True, retire=True):
            pm = (p + NB - 1) % NB
            if scat:
                pltpu.make_async_copy(hw_ref.at[idxs[pm]], rows[pm],
                                      gsem[pm]).wait()
                pltpu.async_copy(rows[pm], agg_sh.at[idxd[pm]], ssem[pm],
                                 add=True)
            if retire:
                p2 = (p + NB - 2) % NB
                pltpu.make_async_copy(rows[p2], agg_sh.at[idxd[p2]],
                                      ssem[p2]).wait()
            pltpu.make_async_copy(src_ref.at[pl.ds(ebase, CHA)],
                                  idxs[p], isem[p]).wait()
            pltpu.make_async_copy(dst_ref.at[pl.ds(ebase, CHA)],
                                  idxd[p], dsem[p]).wait()
            pltpu.async_copy(hw_ref.at[idxs[p]], rows[p], gsem[p])
            if load_next:
                idxload(k + 1, (p + 1) % NB)

        idxload(0, 0)
        slot(0, 0, scat=False, retire=False)
        slot(1, 1, retire=False)
        slot(2, 2)

        nmain = (nchunks - NB - 2) // NB

        @pl.loop(0, nmain)
        def _triple(j):
            a = NB * j + NB
            for p in range(NB):
                slot(a + p, p)

        last = nchunks - 1
        slot(last - 1, (last - 1) % NB)
        slot(last, last % NB, load_next=False)
        pf = last % NB
        pltpu.make_async_copy(hw_ref.at[idxs[pf]], rows[pf],
                              gsem[pf]).wait()
        pltpu.async_copy(rows[pf], agg_sh.at[idxd[pf]], ssem[pf], add=True)
        for q in (last - 1, last):
            pq = q % NB
            pltpu.make_async_copy(rows[pq], agg_sh.at[idxd[pq]],
                                  ssem[pq]).wait()

        plsc.subcore_barrier()
        pltpu.sync_copy(agg_sh.at[pl.ds(r0, rows_pt)],
                        agg_out.at[c, pl.ds(r0, rows_pt)])

    return pl.kernel(body,
                     out_type=jax.ShapeDtypeStruct((NC, npad, w), _bf16),
                     mesh=mesh, scratch_types=scratch,
                     compiler_params=pltpu.CompilerParams(
                         use_tc_tiling_on_sc=False,
                         needs_layout_passes=False))


def _make_score(n, e):
    e_pt = e // NS
    nchunks = e_pt // CH
    assert e_pt % CH == 0

    mesh = plsc.VectorSubcoreMesh(core_axis_name="c", subcore_axis_name="s")

    NB = 3
    assert nchunks % NB == 2 and nchunks >= 2 * NB

    scratch = [
        pltpu.VMEM((e_pt,), _i32),
        pltpu.VMEM((e_pt,), _i32),
        *[pltpu.VMEM((CH, AUGW), _f32) for _ in range(NB)],
        *[pltpu.VMEM((CH,), _f32) for _ in range(NB)],
        pltpu.VMEM((LN, 17), _f32),
        *[pltpu.SemaphoreType.DMA for _ in range(3 * NB)],
    ]

    def body(h2_ref, idx_ref, out_ref, idxs_v, idxd_v, *rest):
        buf = rest[0:NB]
        res = rest[NB:2 * NB]
        scr_v = rest[2 * NB]
        gasem = rest[2 * NB + 1:2 * NB + 1 + NB]
        gbsem = rest[2 * NB + 1 + NB:2 * NB + 1 + 2 * NB]
        osem = rest[2 * NB + 1 + 2 * NB:2 * NB + 1 + 3 * NB]
        c = lax.axis_index("c")
        s = lax.axis_index("s")
        ebase = s * e_pt

        pltpu.sync_copy(idx_ref.at[pl.ds(c * 2 * e + ebase, e_pt)], idxs_v)
        pltpu.sync_copy(idx_ref.at[pl.ds(c * 2 * e + e + ebase, e_pt)],
                        idxd_v)

        def issue_src(k, p):
            pltpu.async_copy(h2_ref.at[idxs_v.at[pl.ds(k * CH, CH)]],
                             buf[p], gasem[p])

        def issue_add(k, p):
            pltpu.make_async_copy(h2_ref.at[idxs_v.at[pl.ds(0, CH)]],
                                  buf[p], gasem[p]).wait()
            pltpu.async_copy(h2_ref.at[idxd_v.at[pl.ds(k * CH, CH)]],
                             buf[p], gbsem[p], add=True)

        def compute(k, p, sync_out, owait):
            pltpu.make_async_copy(h2_ref.at[idxd_v.at[pl.ds(0, CH)]],
                                  buf[p], gbsem[p]).wait()
            if owait:
                pltpu.make_async_copy(
                    res[p], out_ref.at[pl.ds(c * e + ebase, CH)],
                    osem[p]).wait()
            buf_v = buf[p]
            res_v = res[p]
            lane = lax.iota(_i32, LN)

            @pl.loop(0, CH // LN)
            def _grp(g):
                @pl.loop(0, LN)
                def _edge(ii):
                    i = g * LN + ii
                    acc = jnp.zeros((LN,), _f32)
                    for t in range(256 // LN):
                        v = buf_v[i, pl.ds(t * LN, LN)]
                        acc = acc + v * v
                    scr_v[ii, pl.ds(0, LN)] = acc
                rows = lane + g * LN
                nsqv = plsc.load_gather(
                    buf_v, [rows, jnp.full((LN,), 256, _i32)])
                tot = jnp.zeros((LN,), _f32)
                for j in range(LN):
                    tot = tot + plsc.load_gather(
                        scr_v, [lane, jnp.full((LN,), j, _i32)])
                res_v[pl.ds(g * LN, LN)] = (tot - nsqv) * 0.5

            dst = out_ref.at[pl.ds(c * e + ebase + k * CH, CH)]
            if sync_out:
                pltpu.sync_copy(res_v, dst)
            else:
                pltpu.async_copy(res_v, dst, osem[p])

        def slot(k, p, sync_out=False, owait=True, prefetch=True,
                 addpre=True):
            compute(k, p, sync_out, owait)
            if prefetch:
                issue_src(k + NB, p)
            if addpre:
                p2 = (p + 2) % NB
                issue_add(k + 2, p2)

        issue_src(0, 0)
        issue_src(1, 1)
        issue_src(2, 2)
        issue_add(0, 0)
        issue_add(1, 1)
        slot(0, 0, owait=False)
        slot(1, 1, owait=False)
        slot(2, 2, owait=False)

        assert (nchunks - 8) % NB == 0
        nmain = (nchunks - 8) // NB

        @pl.loop(0, nmain)
        def _triple(j):
            a = NB * j + NB
            for p in range(NB):
                slot(a + p, p)

        ka = nchunks - 5
        slot(ka, ka % NB)
        slot(ka + 1, (ka + 1) % NB)
        slot(ka + 2, (ka + 2) % NB, prefetch=False)
        slot(ka + 3, (ka + 3) % NB, sync_out=True, prefetch=False,
             addpre=False)
        slot(ka + 4, (ka + 4) % NB, sync_out=True, prefetch=False,
             addpre=False)
        pr = (ka + 2) % NB
        pltpu.make_async_copy(
            res[pr], out_ref.at[pl.ds(c * e + ebase, CH)], osem[pr]).wait()

    return pl.kernel(
        body,
        out_type=jax.ShapeDtypeStruct((NC * e,), _f32),
        mesh=mesh,
        scratch_types=scratch,
        compiler_params=pltpu.CompilerParams(use_tc_tiling_on_sc=False,
                                             needs_layout_passes=False),
    )


@jax.jit
def kernel(x, edge_index, neg_edge_index, W_self, W_neigh, b):
    n, d = x.shape
    e = edge_index.shape[1]
    r = 1000

    npad = ((n + 127) // 128) * 128
    wcat = jnp.concatenate([W_self, W_neigh], axis=1)
    b2 = b[None, :]
    zw = jnp.zeros((npad // NS, HWW), _bf16)

    agg_k = _make_agg(n, e)
    score_k = _make_score(n, e)

    src, dst = edge_index[0], edge_index[1]
    hs1, hw1 = _mm1(x, wcat, r)
    agg1 = agg_k(hw1, src, dst, zw)
    hs2, hw2 = _mm2(hs1, agg1, b2, wcat, r)
    agg2 = agg_k(hw2, src, dst, zw)
    h2aug = _comb(hs2, agg2, b2, r)

    idx_flat = jnp.concatenate(
        [edge_index.reshape(-1), neg_edge_index.reshape(-1)])
    out = score_k(h2aug, idx_flat)
    return (out[:e, None], out[e:, None])


# speedup vs baseline: 1.1250x; 1.1250x over previous
import functools

import jax
import jax.numpy as jnp
from jax import lax
from jax.experimental import pallas as pl
from jax.experimental.pallas import tpu as pltpu
from jax.experimental.pallas import tpu_sc as plsc

NC = 2
NS = 16
LN = 16
CH = 80
AUGW = 272

_f32 = jnp.float32
_bf16 = jnp.bfloat16
_i32 = jnp.int32


HWW = 288


def _aug(cat, d):
    r = cat.shape[0]
    ones = jnp.ones((r, 1), _f32)
    pad = jnp.zeros((r, HWW - d - 1), _f32)
    return jnp.concatenate([cat[:, d:], ones, pad], axis=1).astype(_bf16)


def _agg_mean(agg_ref, d):
    a = agg_ref[0].astype(_f32) + agg_ref[1].astype(_f32)
    rdeg = 1.0 / jnp.maximum(a[:, d:d + 1], 1.0)
    return a[:, :d] * rdeg


def _mm1_body(x_ref, w_ref, hs_ref, hw_ref):
    cat = jnp.dot(x_ref[...], w_ref[...], preferred_element_type=_f32)
    d = x_ref.shape[1]
    hs_ref[...] = cat[:, :d]
    hw_ref[...] = _aug(cat, d)


def _mm2_body(hs_ref, agg_ref, b_ref, w_ref, hs2_ref, hw2_ref):
    d = hs_ref.shape[1]
    h1 = jnp.maximum(hs_ref[...] + _agg_mean(agg_ref, d) + b_ref[...], 0.0)
    cat = jnp.dot(h1, w_ref[...], preferred_element_type=_f32)
    hs2_ref[...] = cat[:, :d]
    hw2_ref[...] = _aug(cat, d)


def _comb_body(hs_ref, agg_ref, b_ref, out_ref):
    d = hs_ref.shape[1]
    h2 = hs_ref[...] + _agg_mean(agg_ref, d) + b_ref[...]
    nsq = jnp.sum(h2 * h2, axis=1, keepdims=True)
    pad = jnp.zeros((h2.shape[0], AUGW - h2.shape[1] - 1), _f32)
    out_ref[...] = jnp.concatenate([h2, nsq, pad], axis=1)


def _tc_specs(d, r):
    row = pl.BlockSpec((r, d), lambda i: (i, 0))
    hww = pl.BlockSpec((r, HWW), lambda i: (i, 0))
    aggs = pl.BlockSpec((2, r, HWW), lambda i: (0, i, 0))
    bias = pl.BlockSpec((1, d), lambda i: (0, 0))
    wmat = pl.BlockSpec((d, 2 * d), lambda i: (0, 0))
    return row, hww, aggs, bias, wmat


def _mm1(x, wcat, r):
    n, d = x.shape
    row, hww, _, _, wmat = _tc_specs(d, r)
    return pl.pallas_call(
        _mm1_body,
        grid=(n // r,),
        in_specs=[row, wmat],
        out_specs=[row, hww],
        out_shape=[jax.ShapeDtypeStruct((n, d), _f32),
                   jax.ShapeDtypeStruct((n, HWW), _bf16)],
    )(x, wcat)


def _mm2(hs, agg, b2, wcat, r):
    n, d = hs.shape
    row, hww, aggs, bias, wmat = _tc_specs(d, r)
    return pl.pallas_call(
        _mm2_body,
        grid=(n // r,),
        in_specs=[row, aggs, bias, wmat],
        out_specs=[row, hww],
        out_shape=[jax.ShapeDtypeStruct((n, d), _f32),
                   jax.ShapeDtypeStruct((n, HWW), _bf16)],
    )(hs, agg, b2, wcat)


def _comb(hs, agg, b2, r):
    n, d = hs.shape
    row, hww, aggs, bias, _ = _tc_specs(d, r)
    return pl.pallas_call(
        _comb_body,
        grid=(n // r,),
        in_specs=[row, aggs, bias],
        out_specs=pl.BlockSpec((r, AUGW), lambda i: (i, 0)),
        out_shape=jax.ShapeDtypeStruct((n, AUGW), _f32),
    )(hs, agg, b2)


def _make_agg(n, e):
    w = HWW
    CHA = 128
    npad = ((n + 127) // 128) * 128
    rows_pt = npad // NS
    e_pt = e // NC // NS
    rem = e_pt % CHA
    nchunks = e_pt // CHA
    NP = nchunks - ((nchunks - 4) % 2)
    assert rem % 8 == 0 and rem > 0

    mesh = plsc.VectorSubcoreMesh(core_axis_name="c", subcore_axis_name="s")

    NB = 2
    assert (NP - NB - 2) % NB == 0 and NP >= 2 * NB

    scratch = [
        *[pltpu.VMEM((CHA,), _i32) for _ in range(NB)],
        *[pltpu.VMEM((CHA,), _i32) for _ in range(NB)],
        *[pltpu.VMEM((CHA, w), _bf16) for _ in range(NB)],
        pltpu.VMEM((rem,), _i32),
        pltpu.VMEM((rem,), _i32),
        pltpu.VMEM((rem, w), _bf16),
        *[pltpu.SemaphoreType.DMA for _ in range(4 * NB)],
        pltpu.VMEM_SHARED((npad, w), _bf16),
    ]

    def body(hw_ref, src_ref, dst_ref, z_ref, agg_out, *rest):
        idxs = rest[0:NB]
        idxd = rest[NB:2 * NB]
        rows = rest[2 * NB:3 * NB]
        idxs_r, idxd_r, rows_r = rest[3 * NB:3 * NB + 3]
        rest = rest[3 * NB + 3:]
        gsem = rest[0:NB]
        isem = rest[NB:2 * NB]
        dsem = rest[2 * NB:3 * NB]
        ssem = rest[3 * NB:4 * NB]
        agg_sh = rest[4 * NB]

        c = lax.axis_index("c")
        s = lax.axis_index("s")
        r0 = s * rows_pt
        ebase = c * (e // NC) + s * e_pt

        pltpu.sync_copy(z_ref, agg_sh.at[pl.ds(r0, rows_pt)])
        plsc.subcore_barrier()

        def idxload(k, p):
            pltpu.async_copy(src_ref.at[pl.ds(ebase + k * CHA, CHA)],
                             idxs[p], isem[p])
            pltpu.async_copy(dst_ref.at[pl.ds(ebase + k * CHA, CHA)],
                             idxd[p], dsem[p])

        def slot(k, p, load_next=True, scat=True, retire=True):
            pm = (p + NB - 1) % NB
            if scat:
                pltpu.make_async_copy(hw_ref.at[idxs[pm]], rows[pm],
                                      gsem[pm]).wait()
                pltpu.async_copy(rows[pm], agg_sh.at[idxd[pm]], ssem[pm],
                                 add=True)
            if retire:
                p2 = (p + NB - 2) % NB
                pltpu.make_async_copy(rows[p2], agg_sh.at[idxd[p2]],
                                      ssem[p2]).wait()
            pltpu.make_async_copy(src_ref.at[pl.ds(ebase, CHA)],
                                  idxs[p], isem[p]).wait()
            pltpu.make_async_copy(dst_ref.at[pl.ds(ebase, CHA)],
                                  idxd[p], dsem[p]).wait()
            pltpu.async_copy(hw_ref.at[idxs[p]], rows[p], gsem[p])
            if load_next:
                idxload(k + 1, (p + 1) % NB)

        idxload(0, 0)
        slot(0, 0, scat=False, retire=False)
        slot(1, 1, retire=False)

        nmain = (NP - NB - 2) // NB

        @pl.loop(0, nmain)
        def _pair(j):
            a = NB * j + NB
            for p in range(NB):
                slot(a + p, p)

        last = NP - 1
        slot(last - 1, (last - 1) % NB)
        slot(last, last % NB, load_next=False)
        pf = last % NB
        pltpu.make_async_copy(hw_ref.at[idxs[pf]], rows[pf],
                              gsem[pf]).wait()
        pltpu.async_copy(rows[pf], agg_sh.at[idxd[pf]], ssem[pf], add=True)
        for q in (last - 1, last):
            pq = q % NB
            pltpu.make_async_copy(rows[pq], agg_sh.at[idxd[pq]],
                                  ssem[pq]).wait()

        for q in range(NP, nchunks):
            qoff = ebase + q * CHA
            pltpu.sync_copy(src_ref.at[pl.ds(qoff, CHA)], idxs[0])
            pltpu.sync_copy(dst_ref.at[pl.ds(qoff, CHA)], idxd[0])
            pltpu.async_copy(hw_ref.at[idxs[0]], rows[0], gsem[0]).wait()
            pltpu.sync_copy(rows[0], agg_sh.at[idxd[0]], add=True)

        roff = ebase + nchunks * CHA
        pltpu.sync_copy(src_ref.at[pl.ds(roff, rem)], idxs_r)
        pltpu.sync_copy(dst_ref.at[pl.ds(roff, rem)], idxd_r)
        pltpu.async_copy(hw_ref.at[idxs_r], rows_r, gsem[0]).wait()
        pltpu.sync_copy(rows_r, agg_sh.at[idxd_r], add=True)

        plsc.subcore_barrier()
        pltpu.sync_copy(agg_sh.at[pl.ds(r0, rows_pt)],
                        agg_out.at[c, pl.ds(r0, rows_pt)])

    return pl.kernel(body,
                     out_type=jax.ShapeDtypeStruct((NC, npad, w), _bf16),
                     mesh=mesh, scratch_types=scratch,
                     compiler_params=pltpu.CompilerParams(
                         use_tc_tiling_on_sc=False,
                         needs_layout_passes=False))


def _make_score(n, e):
    e_pt = e // NS
    nchunks = e_pt // CH
    assert e_pt % CH == 0

    mesh = plsc.VectorSubcoreMesh(core_axis_name="c", subcore_axis_name="s")

    NB = 3
    assert nchunks % NB == 2 and nchunks >= 2 * NB

    scratch = [
        pltpu.VMEM((e_pt,), _i32),
        pltpu.VMEM((e_pt,), _i32),
        *[pltpu.VMEM((CH, AUGW), _f32) for _ in range(NB)],
        *[pltpu.VMEM((CH,), _f32) for _ in range(NB)],
        pltpu.VMEM((LN, 17), _f32),
        *[pltpu.SemaphoreType.DMA for _ in range(3 * NB)],
    ]

    def body(h2_ref, idx_ref, out_ref, idxs_v, idxd_v, *rest):
        buf = rest[0:NB]
        res = rest[NB:2 * NB]
        scr_v = rest[2 * NB]
        gasem = rest[2 * NB + 1:2 * NB + 1 + NB]
        gbsem = rest[2 * NB + 1 + NB:2 * NB + 1 + 2 * NB]
        osem = rest[2 * NB + 1 + 2 * NB:2 * NB + 1 + 3 * NB]
        c = lax.axis_index("c")
        s = lax.axis_index("s")
        ebase = s * e_pt

        pltpu.sync_copy(idx_ref.at[pl.ds(c * 2 * e + ebase, e_pt)], idxs_v)
        pltpu.sync_copy(idx_ref.at[pl.ds(c * 2 * e + e + ebase, e_pt)],
                        idxd_v)

        def issue_src(k, p):
            pltpu.async_copy(h2_ref.at[idxs_v.at[pl.ds(k * CH, CH)]],
                             buf[p], gasem[p])

        def issue_add(k, p):
            pltpu.make_async_copy(h2_ref.at[idxs_v.at[pl.ds(0, CH)]],
                                  buf[p], gasem[p]).wait()
            pltpu.async_copy(h2_ref.at[idxd_v.at[pl.ds(k * CH, CH)]],
                             buf[p], gbsem[p], add=True)

        def compute(k, p, sync_out, owait):
            pltpu.make_async_copy(h2_ref.at[idxd_v.at[pl.ds(0, CH)]],
                                  buf[p], gbsem[p]).wait()
            if owait:
                pltpu.make_async_copy(
                    res[p], out_ref.at[pl.ds(c * e + ebase, CH)],
                    osem[p]).wait()
            buf_v = buf[p]
            res_v = res[p]
            lane = lax.iota(_i32, LN)

            @pl.loop(0, CH // LN)
            def _grp(g):
                @pl.loop(0, LN)
                def _edge(ii):
                    i = g * LN + ii
                    acc = jnp.zeros((LN,), _f32)
                    for t in range(256 // LN):
                        v = buf_v[i, pl.ds(t * LN, LN)]
                        acc = acc + v * v
                    scr_v[ii, pl.ds(0, LN)] = acc
                rows = lane + g * LN
                nsqv = plsc.load_gather(
                    buf_v, [rows, jnp.full((LN,), 256, _i32)])
                tot = jnp.zeros((LN,), _f32)
                for j in range(LN):
                    tot = tot + plsc.load_gather(
                        scr_v, [lane, jnp.full((LN,), j, _i32)])
                res_v[pl.ds(g * LN, LN)] = (tot - nsqv) * 0.5

            dst = out_ref.at[pl.ds(c * e + ebase + k * CH, CH)]
            if sync_out:
                pltpu.sync_copy(res_v, dst)
            else:
                pltpu.async_copy(res_v, dst, osem[p])

        def slot(k, p, sync_out=False, owait=True, prefetch=True,
                 addpre=True):
            compute(k, p, sync_out, owait)
            if prefetch:
                issue_src(k + NB, p)
            if addpre:
                p2 = (p + 2) % NB
                issue_add(k + 2, p2)

        issue_src(0, 0)
        issue_src(1, 1)
        issue_src(2, 2)
        issue_add(0, 0)
        issue_add(1, 1)
        slot(0, 0, owait=False)
        slot(1, 1, owait=False)
        slot(2, 2, owait=False)

        assert (nchunks - 8) % NB == 0
        nmain = (nchunks - 8) // NB

        @pl.loop(0, nmain)
        def _triple(j):
            a = NB * j + NB
            for p in range(NB):
                slot(a + p, p)

        ka = nchunks - 5
        slot(ka, ka % NB)
        slot(ka + 1, (ka + 1) % NB)
        slot(ka + 2, (ka + 2) % NB, prefetch=False)
        slot(ka + 3, (ka + 3) % NB, sync_out=True, prefetch=False,
             addpre=False)
        slot(ka + 4, (ka + 4) % NB, sync_out=True, prefetch=False,
             addpre=False)
        pr = (ka + 2) % NB
        pltpu.make_async_copy(
            res[pr], out_ref.at[pl.ds(c * e + ebase, CH)], osem[pr]).wait()

    return pl.kernel(
        body,
        out_type=jax.ShapeDtypeStruct((NC * e,), _f32),
        mesh=mesh,
        scratch_types=scratch,
        compiler_params=pltpu.CompilerParams(use_tc_tiling_on_sc=False,
                                             needs_layout_passes=False),
    )


@jax.jit
def kernel(x, edge_index, neg_edge_index, W_self, W_neigh, b):
    n, d = x.shape
    e = edge_index.shape[1]
    r = 1000

    npad = ((n + 127) // 128) * 128
    wcat = jnp.concatenate([W_self, W_neigh], axis=1)
    b2 = b[None, :]
    zw = jnp.zeros((npad // NS, HWW), _bf16)

    agg_k = _make_agg(n, e)
    score_k = _make_score(n, e)

    src, dst = edge_index[0], edge_index[1]
    hs1, hw1 = _mm1(x, wcat, r)
    agg1 = agg_k(hw1, src, dst, zw)
    hs2, hw2 = _mm2(hs1, agg1, b2, wcat, r)
    agg2 = agg_k(hw2, src, dst, zw)
    h2aug = _comb(hs2, agg2, b2, r)

    idx_flat = jnp.concatenate(
        [edge_index.reshape(-1), neg_edge_index.reshape(-1)])
    out = score_k(h2aug, idx_flat)
    return (out[:e, None], out[e:, None])
